# Initial kernel scaffold; baseline (speedup 1.0000x reference)
import jax
import jax.numpy as jnp
from jax.experimental import pallas as pl


def kernel(x, edge_index, gamma, beta, W, b):
    raise NotImplementedError("write your pallas kernel here")


# speedup vs baseline: 19.6889x; 19.6889x over previous
import functools

import jax
import jax.numpy as jnp
from jax import lax
from jax.experimental import pallas as pl
from jax.experimental.pallas import tpu as pltpu
from jax.experimental.pallas import tpu_sc as plsc

N = 10000
D = 128
E = 320000
NC = 2
NS = 16
NW = NC * NS
K = 128
NCHUNK = E // K
JMAX = (NCHUNK + NW - 1) // NW
NPAD = 10240
DEG_PER_TILE = NPAD // NS
ROWS_PER_TILE = NPAD // NS


def _fill_f32(ref, nwords, value):
    v = jnp.full((16,), value, jnp.float32)

    def body(i, c):
        ref[pl.ds(i * 16, 16)] = v
        return c

    lax.fori_loop(0, nwords // 16, body, 0)


@functools.cache
def _sc_kernels():
    mesh = plsc.VectorSubcoreMesh(core_axis_name="c", subcore_axis_name="s",
                                  num_cores=NC, num_subcores=NS)

    @functools.partial(
        pl.kernel,
        out_type=jax.ShapeDtypeStruct((NC * NPAD,), jnp.float32),
        mesh=mesh,
        scratch_types=[
            pltpu.VMEM((K,), jnp.int32),
            pltpu.VMEM((K,), jnp.float32),
            pltpu.VMEM((DEG_PER_TILE,), jnp.float32),
            pltpu.VMEM_SHARED((NPAD,), jnp.float32),
        ],
    )
    def deg_kernel(dst_hbm, deg_hbm, idx_v, ones_v, zero_v, deg_sh):
        cid = lax.axis_index("c")
        sid = lax.axis_index("s")
        wid = sid * NC + cid
        d0 = sid * DEG_PER_TILE
        _fill_f32(zero_v, DEG_PER_TILE, 0.0)
        _fill_f32(ones_v, K, 1.0)
        pltpu.sync_copy(zero_v, deg_sh.at[pl.ds(d0, DEG_PER_TILE)])
        plsc.subcore_barrier()

        def body(j, c):
            chunk = wid + j * NW

            @pl.when(chunk < NCHUNK)
            def _():
                pltpu.sync_copy(dst_hbm.at[pl.ds(chunk * K, K)], idx_v)
                pltpu.sync_copy(ones_v, deg_sh.at[idx_v], add=True)

            return c

        lax.fori_loop(0, JMAX, body, 0)
        plsc.subcore_barrier()
        pltpu.sync_copy(deg_sh.at[pl.ds(d0, DEG_PER_TILE)],
                        deg_hbm.at[pl.ds(cid * NPAD + d0, DEG_PER_TILE)])

    @functools.partial(
        pl.kernel,
        out_type=jax.ShapeDtypeStruct((NC, NPAD, D), jnp.float32),
        mesh=mesh,
        scratch_types=[
            pltpu.VMEM((K,), jnp.int32),
            pltpu.VMEM((K,), jnp.int32),
            pltpu.VMEM((K, D), jnp.float32),
            pltpu.VMEM_SHARED((NPAD, D), jnp.float32),
            pltpu.SemaphoreType.DMA,
        ],
    )
    def msg_kernel(src_hbm, dst_hbm, hw2_hbm, zero_hbm, acc_hbm,
                   sidx, didx, rows, acc_sh, sem):
        cid = lax.axis_index("c")
        sid = lax.axis_index("s")
        wid = sid * NC + cid
        r0 = sid * ROWS_PER_TILE

        @pl.when(cid == 0)
        def _():
            pltpu.sync_copy(hw2_hbm.at[pl.ds(r0, ROWS_PER_TILE)],
                            acc_sh.at[pl.ds(r0, ROWS_PER_TILE)])

        @pl.when(cid != 0)
        def _():
            pltpu.sync_copy(zero_hbm.at[pl.ds(r0, ROWS_PER_TILE)],
                            acc_sh.at[pl.ds(r0, ROWS_PER_TILE)])

        plsc.subcore_barrier()

        def body(j, c):
            chunk = wid + j * NW

            @pl.when(chunk < NCHUNK)
            def _():
                base = chunk * K
                pltpu.sync_copy(src_hbm.at[pl.ds(base, K)], sidx)
                pltpu.sync_copy(dst_hbm.at[pl.ds(base, K)], didx)
                pltpu.async_copy(hw2_hbm.at[sidx], rows, sem).wait()
                pltpu.sync_copy(rows, acc_sh.at[didx], add=True)

            return c

        lax.fori_loop(0, JMAX, body, 0)
        plsc.subcore_barrier()
        pltpu.sync_copy(acc_sh.at[pl.ds(r0, ROWS_PER_TILE)],
                        acc_hbm.at[cid, pl.ds(r0, ROWS_PER_TILE)])

    return deg_kernel, msg_kernel


def _dense_body(x_ref, w_ref, g_ref, bt_ref, degt_ref, hw2_ref, dis_ref):
    x = x_ref[...]
    mean = jnp.mean(x, axis=0, keepdims=True)
    var = jnp.mean((x - mean) ** 2, axis=0, keepdims=True)
    h = (x - mean) * lax.rsqrt(var + 1e-5) * g_ref[...] + bt_ref[...]
    hw = jnp.dot(h, w_ref[...], preferred_element_type=jnp.float32)
    dgt = degt_ref[...]
    dis = lax.rsqrt(dgt[:, 0:1] + dgt[:, 1:2] + 1.0)
    hw2_ref[pl.ds(0, N)] = hw * dis
    dis_ref[...] = dis


_dense_call = pl.pallas_call(
    _dense_body,
    out_shape=[
        jax.ShapeDtypeStruct((NPAD, D), jnp.float32),
        jax.ShapeDtypeStruct((N, 1), jnp.float32),
    ],
)

BR = 1000


def _out_body(acc_ref, dis_ref, b_ref, x_ref, y_ref):
    s = acc_ref[0] + acc_ref[1]
    o = jnp.maximum(s * dis_ref[...] + b_ref[...], 0.0)
    y_ref[...] = jnp.concatenate([o, x_ref[...]], axis=1)


_out_call = pl.pallas_call(
    _out_body,
    grid=(N // BR,),
    in_specs=[
        pl.BlockSpec((NC, BR, D), lambda i: (0, i, 0)),
        pl.BlockSpec((BR, 1), lambda i: (i, 0)),
        pl.BlockSpec((1, D), lambda i: (0, 0)),
        pl.BlockSpec((BR, D), lambda i: (i, 0)),
    ],
    out_specs=pl.BlockSpec((BR, 2 * D), lambda i: (i, 0)),
    out_shape=jax.ShapeDtypeStruct((N, 2 * D), jnp.float32),
)


def kernel(x, edge_index, gamma, beta, W, b):
    deg_call, msg_call = _sc_kernels()
    src = edge_index[0].astype(jnp.int32)
    dst = edge_index[1].astype(jnp.int32)
    deg2 = deg_call(dst).reshape(NC, NPAD)
    degt = jnp.transpose(deg2)[:N]
    hw2, dis = _dense_call(x, W, gamma.reshape(1, D), beta.reshape(1, D), degt)
    zeros = jnp.zeros((NPAD, D), jnp.float32)
    acc = msg_call(src, dst, hw2, zeros)
    return _out_call(acc, dis, b.reshape(1, D), x)


# speedup vs baseline: 26.8979x; 1.3661x over previous
import functools

import jax
import jax.numpy as jnp
from jax import lax
from jax.experimental import pallas as pl
from jax.experimental.pallas import tpu as pltpu
from jax.experimental.pallas import tpu_sc as plsc

N = 10000
D = 128
E = 320000
NC = 2
NS = 16
NW = NC * NS
K = 128
NCHUNK = E // K
JMAX = (NCHUNK + NW - 1) // NW
NB = 4
KM = 64
NCHUNKM = E // KM
JMAXM = (NCHUNKM + NW - 1) // NW
NG = (JMAXM + NB - 1) // NB
NPAD = 10240
DEG_PER_TILE = NPAD // NS
ROWS_PER_TILE = NPAD // NS


def _fill_f32(ref, nwords, value):
    v = jnp.full((16,), value, jnp.float32)

    def body(i, c):
        ref[pl.ds(i * 16, 16)] = v
        return c

    lax.fori_loop(0, nwords // 16, body, 0)


@functools.cache
def _sc_kernels():
    mesh = plsc.VectorSubcoreMesh(core_axis_name="c", subcore_axis_name="s",
                                  num_cores=NC, num_subcores=NS)

    @functools.partial(
        pl.kernel,
        out_type=jax.ShapeDtypeStruct((NC * NPAD,), jnp.float32),
        mesh=mesh,
        scratch_types=[
            pltpu.VMEM((K,), jnp.int32),
            pltpu.VMEM((K,), jnp.float32),
            pltpu.VMEM((DEG_PER_TILE,), jnp.float32),
            pltpu.VMEM_SHARED((NPAD,), jnp.float32),
        ],
    )
    def deg_kernel(dst_hbm, deg_hbm, idx_v, ones_v, zero_v, deg_sh):
        cid = lax.axis_index("c")
        sid = lax.axis_index("s")
        wid = sid * NC + cid
        d0 = sid * DEG_PER_TILE
        _fill_f32(zero_v, DEG_PER_TILE, 0.0)
        _fill_f32(ones_v, K, 1.0)
        pltpu.sync_copy(zero_v, deg_sh.at[pl.ds(d0, DEG_PER_TILE)])
        plsc.subcore_barrier()

        def body(j, c):
            chunk = wid + j * NW

            @pl.when(chunk < NCHUNK)
            def _():
                pltpu.sync_copy(dst_hbm.at[pl.ds(chunk * K, K)], idx_v)
                pltpu.sync_copy(ones_v, deg_sh.at[idx_v], add=True)

            return c

        lax.fori_loop(0, JMAX, body, 0)
        plsc.subcore_barrier()
        pltpu.sync_copy(deg_sh.at[pl.ds(d0, DEG_PER_TILE)],
                        deg_hbm.at[pl.ds(cid * NPAD + d0, DEG_PER_TILE)])

    @functools.partial(
        pl.kernel,
        out_type=jax.ShapeDtypeStruct((NC, NPAD, D), jnp.float32),
        mesh=mesh,
        scratch_types=[
            pltpu.VMEM((2, NB, KM), jnp.int32),
            pltpu.VMEM((2, NB, KM), jnp.int32),
            pltpu.VMEM((NB, KM, D), jnp.float32),
            pltpu.VMEM_SHARED((NPAD, D), jnp.float32),
            pltpu.SemaphoreType.DMA,
            pltpu.SemaphoreType.DMA,
            pltpu.SemaphoreType.DMA,
        ],
    )
    def msg_kernel(src_hbm, dst_hbm, hw2_hbm, zero_hbm, acc_hbm,
                   sidx, didx, rows, acc_sh, sem_i, sem_g, sem_s):
        cid = lax.axis_index("c")
        sid = lax.axis_index("s")
        wid = sid * NC + cid
        r0 = sid * ROWS_PER_TILE

        def chunk_of(g, b):
            return wid + (g * NB + b) * NW

        def issue_idx(g):
            p = lax.rem(g, 2)
            for b in range(NB):
                ch = chunk_of(g, b)

                @pl.when(ch < NCHUNKM)
                def _():
                    base = ch * KM
                    pltpu.async_copy(src_hbm.at[pl.ds(base, KM)],
                                     sidx.at[p, b], sem_i)
                    pltpu.async_copy(dst_hbm.at[pl.ds(base, KM)],
                                     didx.at[p, b], sem_i)

        def wait_idx(g):
            p = lax.rem(g, 2)
            for b in range(NB):
                ch = chunk_of(g, b)

                @pl.when(ch < NCHUNKM)
                def _():
                    base = ch * KM
                    pltpu.make_async_copy(src_hbm.at[pl.ds(base, KM)],
                                          sidx.at[p, b], sem_i).wait()
                    pltpu.make_async_copy(dst_hbm.at[pl.ds(base, KM)],
                                          didx.at[p, b], sem_i).wait()

        def drain_scatters(g, extra_cond):
            p = lax.rem(g, 2)
            for b in range(NB):
                ch = chunk_of(g, b)

                @pl.when(extra_cond & (ch < NCHUNKM))
                def _():
                    pltpu.make_async_copy(rows.at[b],
                                          acc_sh.at[didx.at[p, b]],
                                          sem_s).wait()

        @pl.when(cid == 0)
        def _():
            pltpu.sync_copy(hw2_hbm.at[pl.ds(r0, ROWS_PER_TILE)],
                            acc_sh.at[pl.ds(r0, ROWS_PER_TILE)])

        @pl.when(cid != 0)
        def _():
            pltpu.sync_copy(zero_hbm.at[pl.ds(r0, ROWS_PER_TILE)],
                            acc_sh.at[pl.ds(r0, ROWS_PER_TILE)])

        plsc.subcore_barrier()
        issue_idx(0)

        def body(g, c):
            p = lax.rem(g, 2)
            drain_scatters(g - 1, g > 0)
            wait_idx(g)
            for b in range(NB):
                ch = chunk_of(g, b)

                @pl.when(ch < NCHUNKM)
                def _():
                    pltpu.async_copy(hw2_hbm.at[sidx.at[p, b]],
                                     rows.at[b], sem_g)
            for b in range(NB):
                ch = chunk_of(g, b)

                @pl.when(ch < NCHUNKM)
                def _():
                    pltpu.make_async_copy(hw2_hbm.at[sidx.at[p, b]],
                                          rows.at[b], sem_g).wait()
            issue_idx(g + 1)
            for b in range(NB):
                ch = chunk_of(g, b)

                @pl.when(ch < NCHUNKM)
                def _():
                    pltpu.async_copy(rows.at[b], acc_sh.at[didx.at[p, b]],
                                     sem_s, add=True)
            return c

        lax.fori_loop(0, NG, body, 0)
        drain_scatters(NG - 1, jnp.bool_(True))
        plsc.subcore_barrier()
        pltpu.sync_copy(acc_sh.at[pl.ds(r0, ROWS_PER_TILE)],
                        acc_hbm.at[cid, pl.ds(r0, ROWS_PER_TILE)])

    return deg_kernel, msg_kernel


def _dense_body(x_ref, w_ref, g_ref, bt_ref, degt_ref, hw2_ref, dis_ref):
    x = x_ref[...]
    mean = jnp.mean(x, axis=0, keepdims=True)
    var = jnp.mean((x - mean) ** 2, axis=0, keepdims=True)
    h = (x - mean) * lax.rsqrt(var + 1e-5) * g_ref[...] + bt_ref[...]
    hw = jnp.dot(h, w_ref[...], preferred_element_type=jnp.float32)
    dgt = degt_ref[...]
    dis = lax.rsqrt(dgt[:, 0:1] + dgt[:, 1:2] + 1.0)
    hw2_ref[pl.ds(0, N)] = hw * dis
    dis_ref[...] = dis


_dense_call = pl.pallas_call(
    _dense_body,
    out_shape=[
        jax.ShapeDtypeStruct((NPAD, D), jnp.float32),
        jax.ShapeDtypeStruct((N, 1), jnp.float32),
    ],
)

BR = 1000


def _out_body(acc_ref, dis_ref, b_ref, x_ref, y_ref):
    s = acc_ref[0] + acc_ref[1]
    o = jnp.maximum(s * dis_ref[...] + b_ref[...], 0.0)
    y_ref[...] = jnp.concatenate([o, x_ref[...]], axis=1)


_out_call = pl.pallas_call(
    _out_body,
    grid=(N // BR,),
    in_specs=[
        pl.BlockSpec((NC, BR, D), lambda i: (0, i, 0)),
        pl.BlockSpec((BR, 1), lambda i: (i, 0)),
        pl.BlockSpec((1, D), lambda i: (0, 0)),
        pl.BlockSpec((BR, D), lambda i: (i, 0)),
    ],
    out_specs=pl.BlockSpec((BR, 2 * D), lambda i: (i, 0)),
    out_shape=jax.ShapeDtypeStruct((N, 2 * D), jnp.float32),
)


def kernel(x, edge_index, gamma, beta, W, b):
    deg_call, msg_call = _sc_kernels()
    src = edge_index[0].astype(jnp.int32)
    dst = edge_index[1].astype(jnp.int32)
    deg2 = deg_call(dst).reshape(NC, NPAD)
    degt = jnp.transpose(deg2)[:N]
    hw2, dis = _dense_call(x, W, gamma.reshape(1, D), beta.reshape(1, D), degt)
    zeros = jnp.zeros((NPAD, D), jnp.float32)
    acc = msg_call(src, dst, hw2, zeros)
    return _out_call(acc, dis, b.reshape(1, D), x)


# speedup vs baseline: 39.5275x; 1.4695x over previous
import functools

import jax
import jax.numpy as jnp
from jax import lax
from jax.experimental import pallas as pl
from jax.experimental.pallas import tpu as pltpu
from jax.experimental.pallas import tpu_sc as plsc

N = 10000
D = 128
E = 320000
NC = 2
NS = 16
NW = NC * NS
K = 128
NCHUNK = E // K
JMAX = (NCHUNK + NW - 1) // NW
NB = 4
KM = 64
NCHUNKM = E // KM
JMAXM = (NCHUNKM + NW - 1) // NW
NG = (JMAXM + NB - 1) // NB
NPAD = 10240
DEG_PER_TILE = NPAD // NS
ROWS_PER_TILE = NPAD // NS


def _fill_f32(ref, nwords, value):
    v = jnp.full((16,), value, jnp.float32)

    def body(i, c):
        ref[pl.ds(i * 16, 16)] = v
        return c

    lax.fori_loop(0, nwords // 16, body, 0)


@functools.cache
def _sc_kernels():
    mesh = plsc.VectorSubcoreMesh(core_axis_name="c", subcore_axis_name="s",
                                  num_cores=NC, num_subcores=NS)

    @functools.partial(
        pl.kernel,
        out_type=jax.ShapeDtypeStruct((NC * NPAD,), jnp.float32),
        mesh=mesh,
        scratch_types=[
            pltpu.VMEM((8, K), jnp.int32),
            pltpu.VMEM((K,), jnp.float32),
            pltpu.VMEM((DEG_PER_TILE,), jnp.float32),
            pltpu.VMEM_SHARED((NPAD,), jnp.float32),
            pltpu.SemaphoreType.DMA,
            pltpu.SemaphoreType.DMA,
        ],
    )
    def deg_kernel(dst_hbm, deg_hbm, didx, ones_v, zero_v, deg_sh,
                   sem_i, sem_s):
        cid = lax.axis_index("c")
        sid = lax.axis_index("s")
        wid = sid * NC + cid
        d0 = sid * DEG_PER_TILE
        _fill_f32(zero_v, DEG_PER_TILE, 0.0)
        _fill_f32(ones_v, K, 1.0)
        pltpu.sync_copy(zero_v, deg_sh.at[pl.ds(d0, DEG_PER_TILE)])
        plsc.subcore_barrier()

        def valid(jj):
            return (jj >= 0) & (wid + jj * NW < NCHUNK)

        def issue_idx(jj):
            @pl.when(valid(jj))
            def _():
                base = (wid + jj * NW) * K
                pltpu.async_copy(dst_hbm.at[pl.ds(base, K)],
                                 didx.at[lax.rem(jj, 8)], sem_i)

        def wait_idx(jj):
            @pl.when(valid(jj))
            def _():
                base = (wid + jj * NW) * K
                pltpu.make_async_copy(dst_hbm.at[pl.ds(base, K)],
                                      didx.at[lax.rem(jj, 8)], sem_i).wait()

        def issue_scat(jj):
            @pl.when(valid(jj))
            def _():
                pltpu.async_copy(ones_v, deg_sh.at[didx.at[lax.rem(jj, 8)]],
                                 sem_s, add=True)

        def wait_scat(jj):
            @pl.when(valid(jj))
            def _():
                pltpu.make_async_copy(ones_v,
                                      deg_sh.at[didx.at[lax.rem(jj, 8)]],
                                      sem_s).wait()

        issue_idx(0)
        issue_idx(1)

        def body(j, c):
            wait_scat(j - 4)
            wait_idx(j)
            issue_scat(j)
            issue_idx(j + 2)
            return c

        lax.fori_loop(0, JMAX + 4, body, 0)
        plsc.subcore_barrier()
        pltpu.sync_copy(deg_sh.at[pl.ds(d0, DEG_PER_TILE)],
                        deg_hbm.at[pl.ds(cid * NPAD + d0, DEG_PER_TILE)])

    @functools.partial(
        pl.kernel,
        out_type=jax.ShapeDtypeStruct((NC, NPAD, D), jnp.float32),
        mesh=mesh,
        scratch_types=[
            pltpu.VMEM((8, KM), jnp.int32),
            pltpu.VMEM((8, KM), jnp.int32),
            pltpu.VMEM((4, KM, D), jnp.float32),
            pltpu.VMEM_SHARED((NPAD, D), jnp.float32),
            pltpu.SemaphoreType.DMA,
            pltpu.SemaphoreType.DMA,
            pltpu.SemaphoreType.DMA,
        ],
    )
    def msg_kernel(src_hbm, dst_hbm, hw2_hbm, zero_hbm, acc_hbm,
                   sidx, didx, rows, acc_sh, sem_i, sem_g, sem_s):
        cid = lax.axis_index("c")
        sid = lax.axis_index("s")
        wid = sid * NC + cid
        r0 = sid * ROWS_PER_TILE

        def valid(jj):
            return (jj >= 0) & (wid + jj * NW < NCHUNKM)

        def issue_idx(jj):
            @pl.when(valid(jj))
            def _():
                base = (wid + jj * NW) * KM
                s = lax.rem(jj, 8)
                pltpu.async_copy(src_hbm.at[pl.ds(base, KM)],
                                 sidx.at[s], sem_i)
                pltpu.async_copy(dst_hbm.at[pl.ds(base, KM)],
                                 didx.at[s], sem_i)

        def wait_idx(jj):
            @pl.when(valid(jj))
            def _():
                base = (wid + jj * NW) * KM
                s = lax.rem(jj, 8)
                pltpu.make_async_copy(src_hbm.at[pl.ds(base, KM)],
                                      sidx.at[s], sem_i).wait()
                pltpu.make_async_copy(dst_hbm.at[pl.ds(base, KM)],
                                      didx.at[s], sem_i).wait()

        def issue_gather(jj):
            @pl.when(valid(jj))
            def _():
                pltpu.async_copy(hw2_hbm.at[sidx.at[lax.rem(jj, 8)]],
                                 rows.at[lax.rem(jj, 4)], sem_g)

        def wait_gather(jj):
            @pl.when(valid(jj))
            def _():
                pltpu.make_async_copy(hw2_hbm.at[sidx.at[lax.rem(jj, 8)]],
                                      rows.at[lax.rem(jj, 4)], sem_g).wait()

        def issue_scat(jj):
            @pl.when(valid(jj))
            def _():
                pltpu.async_copy(rows.at[lax.rem(jj, 4)],
                                 acc_sh.at[didx.at[lax.rem(jj, 8)]],
                                 sem_s, add=True)

        def wait_scat(jj):
            @pl.when(valid(jj))
            def _():
                pltpu.make_async_copy(rows.at[lax.rem(jj, 4)],
                                      acc_sh.at[didx.at[lax.rem(jj, 8)]],
                                      sem_s).wait()

        @pl.when(cid == 0)
        def _():
            pltpu.sync_copy(hw2_hbm.at[pl.ds(r0, ROWS_PER_TILE)],
                            acc_sh.at[pl.ds(r0, ROWS_PER_TILE)])

        @pl.when(cid != 0)
        def _():
            pltpu.sync_copy(zero_hbm.at[pl.ds(r0, ROWS_PER_TILE)],
                            acc_sh.at[pl.ds(r0, ROWS_PER_TILE)])

        plsc.subcore_barrier()
        issue_idx(0)
        issue_idx(1)

        def body(j, c):
            wait_scat(j - 4)
            wait_idx(j)
            issue_gather(j)
            wait_gather(j - 2)
            issue_scat(j - 2)
            issue_idx(j + 2)
            return c

        lax.fori_loop(0, JMAXM + 6, body, 0)
        plsc.subcore_barrier()
        pltpu.sync_copy(acc_sh.at[pl.ds(r0, ROWS_PER_TILE)],
                        acc_hbm.at[cid, pl.ds(r0, ROWS_PER_TILE)])

    return deg_kernel, msg_kernel


def _dense_body(x_ref, w_ref, g_ref, bt_ref, degt_ref, hw2_ref, dis_ref):
    x = x_ref[...]
    mean = jnp.mean(x, axis=0, keepdims=True)
    var = jnp.mean((x - mean) ** 2, axis=0, keepdims=True)
    h = (x - mean) * lax.rsqrt(var + 1e-5) * g_ref[...] + bt_ref[...]
    hw = jnp.dot(h, w_ref[...], preferred_element_type=jnp.float32)
    dgt = degt_ref[...]
    dis = lax.rsqrt(dgt[:, 0:1] + dgt[:, 1:2] + 1.0)
    hw2_ref[pl.ds(0, N)] = hw * dis
    dis_ref[...] = dis


_dense_call = pl.pallas_call(
    _dense_body,
    out_shape=[
        jax.ShapeDtypeStruct((NPAD, D), jnp.float32),
        jax.ShapeDtypeStruct((N, 1), jnp.float32),
    ],
)

BR = 1000


def _out_body(acc_ref, dis_ref, b_ref, x_ref, y_ref):
    s = acc_ref[0] + acc_ref[1]
    o = jnp.maximum(s * dis_ref[...] + b_ref[...], 0.0)
    y_ref[...] = jnp.concatenate([o, x_ref[...]], axis=1)


_out_call = pl.pallas_call(
    _out_body,
    grid=(N // BR,),
    in_specs=[
        pl.BlockSpec((NC, BR, D), lambda i: (0, i, 0)),
        pl.BlockSpec((BR, 1), lambda i: (i, 0)),
        pl.BlockSpec((1, D), lambda i: (0, 0)),
        pl.BlockSpec((BR, D), lambda i: (i, 0)),
    ],
    out_specs=pl.BlockSpec((BR, 2 * D), lambda i: (i, 0)),
    out_shape=jax.ShapeDtypeStruct((N, 2 * D), jnp.float32),
)


def kernel(x, edge_index, gamma, beta, W, b):
    deg_call, msg_call = _sc_kernels()
    src = edge_index[0].astype(jnp.int32)
    dst = edge_index[1].astype(jnp.int32)
    deg2 = deg_call(dst).reshape(NC, NPAD)
    degt = jnp.transpose(deg2)[:N]
    hw2, dis = _dense_call(x, W, gamma.reshape(1, D), beta.reshape(1, D), degt)
    zeros = jnp.zeros((NPAD, D), jnp.float32)
    acc = msg_call(src, dst, hw2, zeros)
    return _out_call(acc, dis, b.reshape(1, D), x)


# speedup vs baseline: 41.1116x; 1.0401x over previous
import functools

import jax
import jax.numpy as jnp
from jax import lax
from jax.experimental import pallas as pl
from jax.experimental.pallas import tpu as pltpu
from jax.experimental.pallas import tpu_sc as plsc

N = 10000
D = 128
E = 320000
NC = 2
NS = 16
NW = NC * NS
KD = 256
NCHUNKD = E // KD
JMAXD = (NCHUNKD + NW - 1) // NW
KM = 80
NCHUNKM = E // KM
JMAXM = NCHUNKM // NW
NPAD = 10240
DEG_PER_TILE = NPAD // NS
ROWS_PER_TILE = NPAD // NS


def _fill_f32(ref, nwords, value):
    v = jnp.full((16,), value, jnp.float32)

    def body(i, c):
        ref[pl.ds(i * 16, 16)] = v
        return c

    lax.fori_loop(0, nwords // 16, body, 0)


@functools.cache
def _sc_kernels():
    mesh = plsc.VectorSubcoreMesh(core_axis_name="c", subcore_axis_name="s",
                                  num_cores=NC, num_subcores=NS)

    @functools.partial(
        pl.kernel,
        out_type=jax.ShapeDtypeStruct((NC * NPAD,), jnp.float32),
        mesh=mesh,
        scratch_types=[
            pltpu.VMEM((8, 2, 128), jnp.int32),
            pltpu.VMEM((128,), jnp.float32),
            pltpu.VMEM((DEG_PER_TILE,), jnp.float32),
            pltpu.VMEM_SHARED((NPAD,), jnp.float32),
            pltpu.SemaphoreType.DMA,
            pltpu.SemaphoreType.DMA,
        ],
    )
    def deg_kernel(dst_hbm, deg_hbm, didx, ones_v, zero_v, deg_sh,
                   sem_i, sem_s):
        cid = lax.axis_index("c")
        sid = lax.axis_index("s")
        wid = sid * NC + cid
        d0 = sid * DEG_PER_TILE
        _fill_f32(zero_v, DEG_PER_TILE, 0.0)
        _fill_f32(ones_v, 128, 1.0)
        pltpu.sync_copy(zero_v, deg_sh.at[pl.ds(d0, DEG_PER_TILE)])
        plsc.subcore_barrier()

        def valid(jj):
            return (jj >= 0) & (wid + jj * NW < NCHUNKD)

        def issue_idx(jj):
            @pl.when(valid(jj))
            def _():
                base = (wid + jj * NW) * KD
                s = lax.rem(jj, 8)
                pltpu.async_copy(dst_hbm.at[pl.ds(base, 128)],
                                 didx.at[s, 0], sem_i)
                pltpu.async_copy(dst_hbm.at[pl.ds(base + 128, 128)],
                                 didx.at[s, 1], sem_i)

        def wait_idx(jj):
            @pl.when(valid(jj))
            def _():
                base = (wid + jj * NW) * KD
                s = lax.rem(jj, 8)
                pltpu.make_async_copy(dst_hbm.at[pl.ds(base, 128)],
                                      didx.at[s, 0], sem_i).wait()
                pltpu.make_async_copy(dst_hbm.at[pl.ds(base + 128, 128)],
                                      didx.at[s, 1], sem_i).wait()

        def issue_scat(jj):
            @pl.when(valid(jj))
            def _():
                s = lax.rem(jj, 8)
                pltpu.async_copy(ones_v, deg_sh.at[didx.at[s, 0]],
                                 sem_s, add=True)
                pltpu.async_copy(ones_v, deg_sh.at[didx.at[s, 1]],
                                 sem_s, add=True)

        def wait_scat(jj):
            @pl.when(valid(jj))
            def _():
                s = lax.rem(jj, 8)
                pltpu.make_async_copy(ones_v, deg_sh.at[didx.at[s, 0]],
                                      sem_s).wait()
                pltpu.make_async_copy(ones_v, deg_sh.at[didx.at[s, 1]],
                                      sem_s).wait()

        issue_idx(0)
        issue_idx(1)

        def body(j, c):
            wait_scat(j - 4)
            wait_idx(j)
            issue_scat(j)
            issue_idx(j + 2)
            return c

        lax.fori_loop(0, JMAXD + 4, body, 0)
        plsc.subcore_barrier()
        pltpu.sync_copy(deg_sh.at[pl.ds(d0, DEG_PER_TILE)],
                        deg_hbm.at[pl.ds(cid * NPAD + d0, DEG_PER_TILE)])

    @functools.partial(
        pl.kernel,
        out_type=jax.ShapeDtypeStruct((NC, NPAD, D), jnp.float32),
        mesh=mesh,
        scratch_types=[
            pltpu.VMEM((8, KM), jnp.int32),
            pltpu.VMEM((8, KM), jnp.int32),
            pltpu.VMEM((4, KM, D), jnp.float32),
            pltpu.VMEM_SHARED((NPAD, D), jnp.float32),
            pltpu.SemaphoreType.DMA,
            pltpu.SemaphoreType.DMA,
            pltpu.SemaphoreType.DMA,
        ],
    )
    def msg_kernel(src_hbm, dst_hbm, hw2_hbm, zero_hbm, acc_hbm,
                   sidx, didx, rows, acc_sh, sem_i, sem_g, sem_s):
        cid = lax.axis_index("c")
        sid = lax.axis_index("s")
        wid = sid * NC + cid
        r0 = sid * ROWS_PER_TILE

        def valid(jj):
            return (jj >= 0) & (wid + jj * NW < NCHUNKM)

        def issue_idx(jj):
            @pl.when(valid(jj))
            def _():
                base = (wid + jj * NW) * KM
                s = lax.rem(jj, 8)
                pltpu.async_copy(src_hbm.at[pl.ds(base, KM)],
                                 sidx.at[s], sem_i)
                pltpu.async_copy(dst_hbm.at[pl.ds(base, KM)],
                                 didx.at[s], sem_i)

        def wait_idx(jj):
            @pl.when(valid(jj))
            def _():
                base = (wid + jj * NW) * KM
                s = lax.rem(jj, 8)
                pltpu.make_async_copy(src_hbm.at[pl.ds(base, KM)],
                                      sidx.at[s], sem_i).wait()
                pltpu.make_async_copy(dst_hbm.at[pl.ds(base, KM)],
                                      didx.at[s], sem_i).wait()

        def issue_gather(jj):
            @pl.when(valid(jj))
            def _():
                pltpu.async_copy(hw2_hbm.at[sidx.at[lax.rem(jj, 8)]],
                                 rows.at[lax.rem(jj, 4)], sem_g)

        def wait_gather(jj):
            @pl.when(valid(jj))
            def _():
                pltpu.make_async_copy(hw2_hbm.at[sidx.at[lax.rem(jj, 8)]],
                                      rows.at[lax.rem(jj, 4)], sem_g).wait()

        def issue_scat(jj):
            @pl.when(valid(jj))
            def _():
                pltpu.async_copy(rows.at[lax.rem(jj, 4)],
                                 acc_sh.at[didx.at[lax.rem(jj, 8)]],
                                 sem_s, add=True)

        def wait_scat(jj):
            @pl.when(valid(jj))
            def _():
                pltpu.make_async_copy(rows.at[lax.rem(jj, 4)],
                                      acc_sh.at[didx.at[lax.rem(jj, 8)]],
                                      sem_s).wait()

        issue_idx(0)
        issue_idx(1)

        @pl.when(cid == 0)
        def _():
            pltpu.sync_copy(hw2_hbm.at[pl.ds(r0, ROWS_PER_TILE)],
                            acc_sh.at[pl.ds(r0, ROWS_PER_TILE)])

        @pl.when(cid != 0)
        def _():
            pltpu.sync_copy(zero_hbm.at[pl.ds(r0, ROWS_PER_TILE)],
                            acc_sh.at[pl.ds(r0, ROWS_PER_TILE)])

        plsc.subcore_barrier()

        def body(j, c):
            wait_scat(j - 4)
            wait_idx(j)
            issue_gather(j)
            wait_gather(j - 2)
            issue_scat(j - 2)
            issue_idx(j + 2)
            return c

        lax.fori_loop(0, JMAXM + 6, body, 0)
        plsc.subcore_barrier()
        pltpu.sync_copy(acc_sh.at[pl.ds(r0, ROWS_PER_TILE)],
                        acc_hbm.at[cid, pl.ds(r0, ROWS_PER_TILE)])

    return deg_kernel, msg_kernel


def _dense_body(x_ref, w_ref, g_ref, bt_ref, degt_ref, hw2_ref, dis_ref):
    x = x_ref[...]
    mean = jnp.mean(x, axis=0, keepdims=True)
    var = jnp.mean((x - mean) ** 2, axis=0, keepdims=True)
    h = (x - mean) * lax.rsqrt(var + 1e-5) * g_ref[...] + bt_ref[...]
    hw = jnp.dot(h, w_ref[...], preferred_element_type=jnp.float32)
    dgt = degt_ref[...]
    dis = lax.rsqrt(dgt[:, 0:1] + dgt[:, 1:2] + 1.0)
    hw2_ref[pl.ds(0, N)] = hw * dis
    dis_ref[...] = dis


_dense_call = pl.pallas_call(
    _dense_body,
    out_shape=[
        jax.ShapeDtypeStruct((NPAD, D), jnp.float32),
        jax.ShapeDtypeStruct((N, 1), jnp.float32),
    ],
)

BR = 1000


def _out_body(acc_ref, dis_ref, b_ref, x_ref, y_ref):
    s = acc_ref[0] + acc_ref[1]
    o = jnp.maximum(s * dis_ref[...] + b_ref[...], 0.0)
    y_ref[...] = jnp.concatenate([o, x_ref[...]], axis=1)


_out_call = pl.pallas_call(
    _out_body,
    grid=(N // BR,),
    in_specs=[
        pl.BlockSpec((NC, BR, D), lambda i: (0, i, 0)),
        pl.BlockSpec((BR, 1), lambda i: (i, 0)),
        pl.BlockSpec((1, D), lambda i: (0, 0)),
        pl.BlockSpec((BR, D), lambda i: (i, 0)),
    ],
    out_specs=pl.BlockSpec((BR, 2 * D), lambda i: (i, 0)),
    out_shape=jax.ShapeDtypeStruct((N, 2 * D), jnp.float32),
)


def kernel(x, edge_index, gamma, beta, W, b):
    deg_call, msg_call = _sc_kernels()
    src = edge_index[0].astype(jnp.int32)
    dst = edge_index[1].astype(jnp.int32)
    deg2 = deg_call(dst).reshape(NC, NPAD)
    degt = jnp.transpose(deg2)[:N]
    hw2, dis = _dense_call(x, W, gamma.reshape(1, D), beta.reshape(1, D), degt)
    zeros = jnp.zeros((NPAD, D), jnp.float32)
    acc = msg_call(src, dst, hw2, zeros)
    return _out_call(acc, dis, b.reshape(1, D), x)


# speedup vs baseline: 42.3570x; 1.0303x over previous
import functools

import jax
import jax.numpy as jnp
from jax import lax
from jax.experimental import pallas as pl
from jax.experimental.pallas import tpu as pltpu
from jax.experimental.pallas import tpu_sc as plsc

N = 10000
D = 128
E = 320000
NC = 2
NS = 16
NW = NC * NS
KD = 256
NCHUNKD = E // KD
JMAXD = (NCHUNKD + NW - 1) // NW
KM = 80
NCHUNKM = E // KM
JMAXM = NCHUNKM // NW
NPAD = 10240
DEG_PER_TILE = NPAD // NS
ROWS_PER_TILE = NPAD // NS


def _fill_f32(ref, nwords, value):
    v = jnp.full((16,), value, jnp.float32)

    def body(i, c):
        ref[pl.ds(i * 16, 16)] = v
        return c

    lax.fori_loop(0, nwords // 16, body, 0)


@functools.cache
def _sc_kernels():
    mesh = plsc.VectorSubcoreMesh(core_axis_name="c", subcore_axis_name="s",
                                  num_cores=NC, num_subcores=NS)

    @functools.partial(
        pl.kernel,
        out_type=jax.ShapeDtypeStruct((NC * NPAD,), jnp.float32),
        mesh=mesh,
        scratch_types=[
            pltpu.VMEM((8, 2, 128), jnp.int32),
            pltpu.VMEM((128,), jnp.float32),
            pltpu.VMEM((DEG_PER_TILE,), jnp.float32),
            pltpu.VMEM_SHARED((NPAD,), jnp.float32),
            pltpu.SemaphoreType.DMA,
            pltpu.SemaphoreType.DMA,
        ],
    )
    def deg_kernel(dst_hbm, deg_hbm, didx, ones_v, zero_v, deg_sh,
                   sem_i, sem_s):
        cid = lax.axis_index("c")
        sid = lax.axis_index("s")
        wid = sid * NC + cid
        d0 = sid * DEG_PER_TILE
        _fill_f32(zero_v, DEG_PER_TILE, 0.0)
        _fill_f32(ones_v, 128, 1.0)
        pltpu.sync_copy(zero_v, deg_sh.at[pl.ds(d0, DEG_PER_TILE)])
        plsc.subcore_barrier()

        def valid(jj):
            return (jj >= 0) & (wid + jj * NW < NCHUNKD)

        def issue_idx(jj):
            @pl.when(valid(jj))
            def _():
                base = (wid + jj * NW) * KD
                s = lax.rem(jj, 8)
                pltpu.async_copy(dst_hbm.at[pl.ds(base, 128)],
                                 didx.at[s, 0], sem_i)
                pltpu.async_copy(dst_hbm.at[pl.ds(base + 128, 128)],
                                 didx.at[s, 1], sem_i)

        def wait_idx(jj):
            @pl.when(valid(jj))
            def _():
                base = (wid + jj * NW) * KD
                s = lax.rem(jj, 8)
                pltpu.make_async_copy(dst_hbm.at[pl.ds(base, 128)],
                                      didx.at[s, 0], sem_i).wait()
                pltpu.make_async_copy(dst_hbm.at[pl.ds(base + 128, 128)],
                                      didx.at[s, 1], sem_i).wait()

        def issue_scat(jj):
            @pl.when(valid(jj))
            def _():
                s = lax.rem(jj, 8)
                pltpu.async_copy(ones_v, deg_sh.at[didx.at[s, 0]],
                                 sem_s, add=True)
                pltpu.async_copy(ones_v, deg_sh.at[didx.at[s, 1]],
                                 sem_s, add=True)

        def wait_scat(jj):
            @pl.when(valid(jj))
            def _():
                s = lax.rem(jj, 8)
                pltpu.make_async_copy(ones_v, deg_sh.at[didx.at[s, 0]],
                                      sem_s).wait()
                pltpu.make_async_copy(ones_v, deg_sh.at[didx.at[s, 1]],
                                      sem_s).wait()

        issue_idx(0)
        issue_idx(1)

        def body(j, c):
            wait_scat(j - 4)
            wait_idx(j)
            issue_scat(j)
            issue_idx(j + 2)
            return c

        lax.fori_loop(0, JMAXD + 4, body, 0)
        plsc.subcore_barrier()
        pltpu.sync_copy(deg_sh.at[pl.ds(d0, DEG_PER_TILE)],
                        deg_hbm.at[pl.ds(cid * NPAD + d0, DEG_PER_TILE)])

    @functools.partial(
        pl.kernel,
        out_type=jax.ShapeDtypeStruct((NC, NPAD, D), jnp.float32),
        mesh=mesh,
        scratch_types=[
            pltpu.VMEM((8, KM), jnp.int32),
            pltpu.VMEM((8, KM), jnp.int32),
            pltpu.VMEM((4, KM, D), jnp.float32),
            pltpu.VMEM_SHARED((NPAD, D), jnp.float32),
            pltpu.SemaphoreType.DMA,
            pltpu.SemaphoreType.DMA,
            pltpu.SemaphoreType.DMA,
        ],
    )
    def msg_kernel(src_hbm, dst_hbm, hw2_hbm, zero_hbm, acc_hbm,
                   sidx, didx, rows, acc_sh, sem_i, sem_g, sem_s):
        cid = lax.axis_index("c")
        sid = lax.axis_index("s")
        wid = sid * NC + cid
        r0 = sid * ROWS_PER_TILE

        def valid(jj):
            return (jj >= 0) & (wid + jj * NW < NCHUNKM)

        def issue_idx(jj):
            @pl.when(valid(jj))
            def _():
                base = (wid + jj * NW) * KM
                s = lax.rem(jj, 8)
                pltpu.async_copy(src_hbm.at[pl.ds(base, KM)],
                                 sidx.at[s], sem_i)
                pltpu.async_copy(dst_hbm.at[pl.ds(base, KM)],
                                 didx.at[s], sem_i)

        def wait_idx(jj):
            @pl.when(valid(jj))
            def _():
                base = (wid + jj * NW) * KM
                s = lax.rem(jj, 8)
                pltpu.make_async_copy(src_hbm.at[pl.ds(base, KM)],
                                      sidx.at[s], sem_i).wait()
                pltpu.make_async_copy(dst_hbm.at[pl.ds(base, KM)],
                                      didx.at[s], sem_i).wait()

        def issue_gather(jj):
            @pl.when(valid(jj))
            def _():
                pltpu.async_copy(hw2_hbm.at[sidx.at[lax.rem(jj, 8)]],
                                 rows.at[lax.rem(jj, 4)], sem_g)

        def wait_gather(jj):
            @pl.when(valid(jj))
            def _():
                pltpu.make_async_copy(hw2_hbm.at[sidx.at[lax.rem(jj, 8)]],
                                      rows.at[lax.rem(jj, 4)], sem_g).wait()

        def issue_scat(jj):
            @pl.when(valid(jj))
            def _():
                pltpu.async_copy(rows.at[lax.rem(jj, 4)],
                                 acc_sh.at[didx.at[lax.rem(jj, 8)]],
                                 sem_s, add=True)

        def wait_scat(jj):
            @pl.when(valid(jj))
            def _():
                pltpu.make_async_copy(rows.at[lax.rem(jj, 4)],
                                      acc_sh.at[didx.at[lax.rem(jj, 8)]],
                                      sem_s).wait()

        issue_idx(0)
        issue_idx(1)

        @pl.when(cid == 0)
        def _():
            pltpu.sync_copy(hw2_hbm.at[pl.ds(r0, ROWS_PER_TILE)],
                            acc_sh.at[pl.ds(r0, ROWS_PER_TILE)])

        @pl.when(cid != 0)
        def _():
            pltpu.sync_copy(zero_hbm.at[pl.ds(r0, ROWS_PER_TILE)],
                            acc_sh.at[pl.ds(r0, ROWS_PER_TILE)])

        plsc.subcore_barrier()

        def body(j, c):
            wait_scat(j - 4)
            wait_idx(j)
            issue_gather(j)
            wait_gather(j - 2)
            issue_scat(j - 2)
            issue_idx(j + 2)
            return c

        lax.fori_loop(0, JMAXM + 6, body, 0)
        plsc.subcore_barrier()
        pltpu.sync_copy(acc_sh.at[pl.ds(r0, ROWS_PER_TILE)],
                        acc_hbm.at[cid, pl.ds(r0, ROWS_PER_TILE)])

    return deg_kernel, msg_kernel


def _dense_body(x_ref, w_ref, g_ref, bt_ref, degp_ref, hw2_ref, dis_ref):
    x = x_ref[...]
    mean = jnp.mean(x, axis=0, keepdims=True)
    var = jnp.mean((x - mean) ** 2, axis=0, keepdims=True)
    h = (x - mean) * lax.rsqrt(var + 1e-5) * g_ref[...] + bt_ref[...]
    hw = jnp.dot(h, w_ref[...], preferred_element_type=jnp.float32)
    degp = degp_ref[...]
    deg_row = degp[0:1, :] + degp[1:2, :] + 1.0
    dis = lax.rsqrt(jnp.transpose(deg_row)[:N])
    hw2_ref[pl.ds(0, N)] = hw * dis
    dis_ref[...] = dis


_dense_call = pl.pallas_call(
    _dense_body,
    out_shape=[
        jax.ShapeDtypeStruct((NPAD, D), jnp.float32),
        jax.ShapeDtypeStruct((N, 1), jnp.float32),
    ],
)

BR = 1000


def _out_body(acc_ref, dis_ref, b_ref, x_ref, y_ref):
    s = acc_ref[0] + acc_ref[1]
    o = jnp.maximum(s * dis_ref[...] + b_ref[...], 0.0)
    y_ref[...] = jnp.concatenate([o, x_ref[...]], axis=1)


_out_call = pl.pallas_call(
    _out_body,
    grid=(N // BR,),
    in_specs=[
        pl.BlockSpec((NC, BR, D), lambda i: (0, i, 0)),
        pl.BlockSpec((BR, 1), lambda i: (i, 0)),
        pl.BlockSpec((1, D), lambda i: (0, 0)),
        pl.BlockSpec((BR, D), lambda i: (i, 0)),
    ],
    out_specs=pl.BlockSpec((BR, 2 * D), lambda i: (i, 0)),
    out_shape=jax.ShapeDtypeStruct((N, 2 * D), jnp.float32),
)


def kernel(x, edge_index, gamma, beta, W, b):
    deg_call, msg_call = _sc_kernels()
    src = edge_index[0].astype(jnp.int32)
    dst = edge_index[1].astype(jnp.int32)
    deg2 = deg_call(dst).reshape(NC, NPAD)
    hw2, dis = _dense_call(x, W, gamma.reshape(1, D), beta.reshape(1, D), deg2)
    zeros = jnp.zeros((NPAD, D), jnp.float32)
    acc = msg_call(src, dst, hw2, zeros)
    return _out_call(acc, dis, b.reshape(1, D), x)


# speedup vs baseline: 46.1586x; 1.0898x over previous
import functools

import jax
import jax.numpy as jnp
from jax import lax
from jax.experimental import pallas as pl
from jax.experimental.pallas import tpu as pltpu
from jax.experimental.pallas import tpu_sc as plsc

N = 10000
D = 128
E = 320000
NC = 2
NS = 16
NW = NC * NS
KD = 256
NCHUNKD = E // KD
JMAXD = (NCHUNKD + NW - 1) // NW
KM = 80
NCHUNKM = E // KM
JMAXM = NCHUNKM // NW
NPAD = 10240
DEG_PER_TILE = NPAD // NS
ROWS_PER_TILE = NPAD // NS


def _fill_f32(ref, nwords, value):
    v = jnp.full((16,), value, jnp.float32)

    def body(i, c):
        ref[pl.ds(i * 16, 16)] = v
        return c

    lax.fori_loop(0, nwords // 16, body, 0)


@functools.cache
def _sc_kernels():
    mesh = plsc.VectorSubcoreMesh(core_axis_name="c", subcore_axis_name="s",
                                  num_cores=NC, num_subcores=NS)

    @functools.partial(
        pl.kernel,
        out_type=jax.ShapeDtypeStruct((NC * NPAD,), jnp.float32),
        mesh=mesh,
        scratch_types=[
            pltpu.VMEM((8, 2, 128), jnp.int32),
            pltpu.VMEM((128,), jnp.float32),
            pltpu.VMEM((DEG_PER_TILE,), jnp.float32),
            pltpu.VMEM_SHARED((NPAD,), jnp.float32),
            pltpu.SemaphoreType.DMA,
            pltpu.SemaphoreType.DMA,
        ],
    )
    def deg_kernel(ei_hbm, deg_hbm, didx, ones_v, zero_v, deg_sh,
                   sem_i, sem_s):
        cid = lax.axis_index("c")
        sid = lax.axis_index("s")
        wid = sid * NC + cid
        d0 = sid * DEG_PER_TILE
        _fill_f32(zero_v, DEG_PER_TILE, 0.0)
        _fill_f32(ones_v, 128, 1.0)
        pltpu.sync_copy(zero_v, deg_sh.at[pl.ds(d0, DEG_PER_TILE)])
        plsc.subcore_barrier()

        def valid(jj):
            return (jj >= 0) & (wid + jj * NW < NCHUNKD)

        def issue_idx(jj):
            @pl.when(valid(jj))
            def _():
                base = E + (wid + jj * NW) * KD
                s = lax.rem(jj, 8)
                pltpu.async_copy(ei_hbm.at[pl.ds(base, 128)],
                                 didx.at[s, 0], sem_i)
                pltpu.async_copy(ei_hbm.at[pl.ds(base + 128, 128)],
                                 didx.at[s, 1], sem_i)

        def wait_idx(jj):
            @pl.when(valid(jj))
            def _():
                base = E + (wid + jj * NW) * KD
                s = lax.rem(jj, 8)
                pltpu.make_async_copy(ei_hbm.at[pl.ds(base, 128)],
                                      didx.at[s, 0], sem_i).wait()
                pltpu.make_async_copy(ei_hbm.at[pl.ds(base + 128, 128)],
                                      didx.at[s, 1], sem_i).wait()

        def issue_scat(jj):
            @pl.when(valid(jj))
            def _():
                s = lax.rem(jj, 8)
                pltpu.async_copy(ones_v, deg_sh.at[didx.at[s, 0]],
                                 sem_s, add=True)
                pltpu.async_copy(ones_v, deg_sh.at[didx.at[s, 1]],
                                 sem_s, add=True)

        def wait_scat(jj):
            @pl.when(valid(jj))
            def _():
                s = lax.rem(jj, 8)
                pltpu.make_async_copy(ones_v, deg_sh.at[didx.at[s, 0]],
                                      sem_s).wait()
                pltpu.make_async_copy(ones_v, deg_sh.at[didx.at[s, 1]],
                                      sem_s).wait()

        issue_idx(0)
        issue_idx(1)

        def body(j, c):
            wait_scat(j - 4)
            wait_idx(j)
            issue_scat(j)
            issue_idx(j + 2)
            return c

        lax.fori_loop(0, JMAXD + 4, body, 0)
        plsc.subcore_barrier()
        pltpu.sync_copy(deg_sh.at[pl.ds(d0, DEG_PER_TILE)],
                        deg_hbm.at[pl.ds(cid * NPAD + d0, DEG_PER_TILE)])

    @functools.partial(
        pl.kernel,
        out_type=jax.ShapeDtypeStruct((NC, NPAD, D), jnp.float32),
        mesh=mesh,
        scratch_types=[
            pltpu.VMEM((8, KM), jnp.int32),
            pltpu.VMEM((8, KM), jnp.int32),
            pltpu.VMEM((4, KM, D), jnp.float32),
            pltpu.VMEM_SHARED((NPAD, D), jnp.float32),
            pltpu.SemaphoreType.DMA,
            pltpu.SemaphoreType.DMA,
            pltpu.SemaphoreType.DMA,
        ],
    )
    def msg_kernel(ei_hbm, hw2_hbm, acc_hbm,
                   sidx, didx, rows, acc_sh, sem_i, sem_g, sem_s):
        cid = lax.axis_index("c")
        sid = lax.axis_index("s")
        wid = sid * NC + cid
        r0 = sid * ROWS_PER_TILE

        def valid(jj):
            return (jj >= 0) & (wid + jj * NW < NCHUNKM)

        def issue_idx(jj):
            @pl.when(valid(jj))
            def _():
                base = (wid + jj * NW) * KM
                s = lax.rem(jj, 8)
                pltpu.async_copy(ei_hbm.at[pl.ds(base, KM)],
                                 sidx.at[s], sem_i)
                pltpu.async_copy(ei_hbm.at[pl.ds(E + base, KM)],
                                 didx.at[s], sem_i)

        def wait_idx(jj):
            @pl.when(valid(jj))
            def _():
                base = (wid + jj * NW) * KM
                s = lax.rem(jj, 8)
                pltpu.make_async_copy(ei_hbm.at[pl.ds(base, KM)],
                                      sidx.at[s], sem_i).wait()
                pltpu.make_async_copy(ei_hbm.at[pl.ds(E + base, KM)],
                                      didx.at[s], sem_i).wait()

        def issue_gather(jj):
            @pl.when(valid(jj))
            def _():
                pltpu.async_copy(hw2_hbm.at[sidx.at[lax.rem(jj, 8)]],
                                 rows.at[lax.rem(jj, 4)], sem_g)

        def wait_gather(jj):
            @pl.when(valid(jj))
            def _():
                pltpu.make_async_copy(hw2_hbm.at[sidx.at[lax.rem(jj, 8)]],
                                      rows.at[lax.rem(jj, 4)], sem_g).wait()

        def issue_scat(jj):
            @pl.when(valid(jj))
            def _():
                pltpu.async_copy(rows.at[lax.rem(jj, 4)],
                                 acc_sh.at[didx.at[lax.rem(jj, 8)]],
                                 sem_s, add=True)

        def wait_scat(jj):
            @pl.when(valid(jj))
            def _():
                pltpu.make_async_copy(rows.at[lax.rem(jj, 4)],
                                      acc_sh.at[didx.at[lax.rem(jj, 8)]],
                                      sem_s).wait()

        issue_idx(0)
        issue_idx(1)

        @pl.when(cid == 0)
        def _():
            pltpu.sync_copy(hw2_hbm.at[pl.ds(r0, ROWS_PER_TILE)],
                            acc_sh.at[pl.ds(r0, ROWS_PER_TILE)])

        @pl.when(cid != 0)
        def _():
            zv = jnp.zeros((16,), jnp.float32)

            def zb(i, c):
                rows[0, i // 8, pl.ds(lax.rem(i, 8) * 16, 16)] = zv
                return c

            lax.fori_loop(0, KM * 8, zb, 0)
            for t in range(NPAD // NS // KM):
                pltpu.sync_copy(rows.at[0],
                                acc_sh.at[pl.ds(r0 + t * KM, KM)])

        plsc.subcore_barrier()

        def body(j, c):
            wait_scat(j - 4)
            wait_idx(j)
            issue_gather(j)
            wait_gather(j - 2)
            issue_scat(j - 2)
            issue_idx(j + 2)
            return c

        lax.fori_loop(0, JMAXM + 6, body, 0)
        plsc.subcore_barrier()
        pltpu.sync_copy(acc_sh.at[pl.ds(r0, ROWS_PER_TILE)],
                        acc_hbm.at[cid, pl.ds(r0, ROWS_PER_TILE)])

    return deg_kernel, msg_kernel


def _dense_body(x_ref, w_ref, g_ref, bt_ref, degf_ref, hw2_ref, dis_ref):
    x = x_ref[...]
    mean = jnp.mean(x, axis=0, keepdims=True)
    var = jnp.mean((x - mean) ** 2, axis=0, keepdims=True)
    h = (x - mean) * lax.rsqrt(var + 1e-5) * g_ref[...] + bt_ref[...]
    hw = jnp.dot(h, w_ref[...], preferred_element_type=jnp.float32)
    degf = degf_ref[...]
    deg_row = (degf[:NPAD] + degf[NPAD:] + 1.0).reshape(1, NPAD)
    dis = lax.rsqrt(jnp.transpose(deg_row)[:N])
    hw2_ref[pl.ds(0, N)] = hw * dis
    dis_ref[...] = dis


_dense_call = pl.pallas_call(
    _dense_body,
    out_shape=[
        jax.ShapeDtypeStruct((NPAD, D), jnp.float32),
        jax.ShapeDtypeStruct((N, 1), jnp.float32),
    ],
)

BR = 1000


def _out_body(acc_ref, dis_ref, b_ref, x_ref, y_ref):
    s = acc_ref[0] + acc_ref[1]
    o = jnp.maximum(s * dis_ref[...] + b_ref[...], 0.0)
    y_ref[...] = jnp.concatenate([o, x_ref[...]], axis=1)


_out_call = pl.pallas_call(
    _out_body,
    grid=(N // BR,),
    in_specs=[
        pl.BlockSpec((NC, BR, D), lambda i: (0, i, 0)),
        pl.BlockSpec((BR, 1), lambda i: (i, 0)),
        pl.BlockSpec((D,), lambda i: (0,)),
        pl.BlockSpec((BR, D), lambda i: (i, 0)),
    ],
    out_specs=pl.BlockSpec((BR, 2 * D), lambda i: (i, 0)),
    out_shape=jax.ShapeDtypeStruct((N, 2 * D), jnp.float32),
)


def kernel(x, edge_index, gamma, beta, W, b):
    deg_call, msg_call = _sc_kernels()
    ei = edge_index.astype(jnp.int32).reshape(2 * E)
    degf = deg_call(ei)
    hw2, dis = _dense_call(x, W, gamma, beta, degf)
    acc = msg_call(ei, hw2)
    return _out_call(acc, dis, b, x)


# speedup vs baseline: 46.6978x; 1.0117x over previous
import functools

import jax
import jax.numpy as jnp
from jax import lax
from jax.experimental import pallas as pl
from jax.experimental.pallas import tpu as pltpu
from jax.experimental.pallas import tpu_sc as plsc

N = 10000
D = 128
E = 320000
NC = 2
NS = 16
NW = NC * NS
KD = 256
NCHUNKD = E // KD
JMAXD = (NCHUNKD + NW - 1) // NW
KM = 80
NCHUNKM = E // KM
JMAXM = NCHUNKM // NW
NPAD = 10240
DEG_PER_TILE = NPAD // NS
ROWS_PER_TILE = NPAD // NS


def _fill_f32(ref, nwords, value):
    v = jnp.full((16,), value, jnp.float32)

    def body(i, c):
        ref[pl.ds(i * 16, 16)] = v
        return c

    lax.fori_loop(0, nwords // 16, body, 0)


@functools.cache
def _sc_kernels():
    mesh = plsc.VectorSubcoreMesh(core_axis_name="c", subcore_axis_name="s",
                                  num_cores=NC, num_subcores=NS)

    @functools.partial(
        pl.kernel,
        out_type=jax.ShapeDtypeStruct((NC * NPAD,), jnp.float32),
        mesh=mesh,
        scratch_types=[
            pltpu.VMEM((8, 2, 128), jnp.int32),
            pltpu.VMEM((128,), jnp.float32),
            pltpu.VMEM((DEG_PER_TILE,), jnp.float32),
            pltpu.VMEM_SHARED((NPAD,), jnp.float32),
            pltpu.SemaphoreType.DMA,
            pltpu.SemaphoreType.DMA,
        ],
    )
    def deg_kernel(ei_hbm, deg_hbm, didx, ones_v, zero_v, deg_sh,
                   sem_i, sem_s):
        cid = lax.axis_index("c")
        sid = lax.axis_index("s")
        wid = sid * NC + cid
        d0 = sid * DEG_PER_TILE
        _fill_f32(zero_v, DEG_PER_TILE, 0.0)
        _fill_f32(ones_v, 128, 1.0)
        pltpu.sync_copy(zero_v, deg_sh.at[pl.ds(d0, DEG_PER_TILE)])
        plsc.subcore_barrier()

        def valid(jj):
            return (jj >= 0) & (wid + jj * NW < NCHUNKD)

        def issue_idx(jj):
            @pl.when(valid(jj))
            def _():
                base = E + (wid + jj * NW) * KD
                s = lax.rem(jj, 8)
                pltpu.async_copy(ei_hbm.at[pl.ds(base, 128)],
                                 didx.at[s, 0], sem_i)
                pltpu.async_copy(ei_hbm.at[pl.ds(base + 128, 128)],
                                 didx.at[s, 1], sem_i)

        def wait_idx(jj):
            @pl.when(valid(jj))
            def _():
                base = E + (wid + jj * NW) * KD
                s = lax.rem(jj, 8)
                pltpu.make_async_copy(ei_hbm.at[pl.ds(base, 128)],
                                      didx.at[s, 0], sem_i).wait()
                pltpu.make_async_copy(ei_hbm.at[pl.ds(base + 128, 128)],
                                      didx.at[s, 1], sem_i).wait()

        def issue_scat(jj):
            @pl.when(valid(jj))
            def _():
                s = lax.rem(jj, 8)
                pltpu.async_copy(ones_v, deg_sh.at[didx.at[s, 0]],
                                 sem_s, add=True)
                pltpu.async_copy(ones_v, deg_sh.at[didx.at[s, 1]],
                                 sem_s, add=True)

        def wait_scat(jj):
            @pl.when(valid(jj))
            def _():
                s = lax.rem(jj, 8)
                pltpu.make_async_copy(ones_v, deg_sh.at[didx.at[s, 0]],
                                      sem_s).wait()
                pltpu.make_async_copy(ones_v, deg_sh.at[didx.at[s, 1]],
                                      sem_s).wait()

        issue_idx(0)
        issue_idx(1)

        def body(j, c):
            wait_scat(j - 4)
            wait_idx(j)
            issue_scat(j)
            issue_idx(j + 2)
            return c

        lax.fori_loop(0, JMAXD + 4, body, 0)
        plsc.subcore_barrier()
        pltpu.sync_copy(deg_sh.at[pl.ds(d0, DEG_PER_TILE)],
                        deg_hbm.at[pl.ds(cid * NPAD + d0, DEG_PER_TILE)])

    @functools.partial(
        pl.kernel,
        out_type=jax.ShapeDtypeStruct((NC, NPAD, D), jnp.float32),
        mesh=mesh,
        scratch_types=[
            pltpu.VMEM((8, KM), jnp.int32),
            pltpu.VMEM((8, KM), jnp.int32),
            pltpu.VMEM((4, KM, D), jnp.float32),
            pltpu.VMEM_SHARED((NPAD, D), jnp.float32),
            pltpu.SemaphoreType.DMA,
            pltpu.SemaphoreType.DMA,
            pltpu.SemaphoreType.DMA,
        ],
    )
    def msg_kernel(ei_hbm, hw2_hbm, acc_hbm,
                   sidx, didx, rows, acc_sh, sem_i, sem_g, sem_s):
        cid = lax.axis_index("c")
        sid = lax.axis_index("s")
        wid = sid * NC + cid
        r0 = sid * ROWS_PER_TILE

        def valid(jj):
            return (jj >= 0) & (wid + jj * NW < NCHUNKM)

        def issue_idx(jj):
            @pl.when(valid(jj))
            def _():
                base = (wid + jj * NW) * KM
                s = lax.rem(jj, 8)
                pltpu.async_copy(ei_hbm.at[pl.ds(base, KM)],
                                 sidx.at[s], sem_i)
                pltpu.async_copy(ei_hbm.at[pl.ds(E + base, KM)],
                                 didx.at[s], sem_i)

        def wait_idx(jj):
            @pl.when(valid(jj))
            def _():
                base = (wid + jj * NW) * KM
                s = lax.rem(jj, 8)
                pltpu.make_async_copy(ei_hbm.at[pl.ds(base, KM)],
                                      sidx.at[s], sem_i).wait()
                pltpu.make_async_copy(ei_hbm.at[pl.ds(E + base, KM)],
                                      didx.at[s], sem_i).wait()

        def issue_gather(jj):
            @pl.when(valid(jj))
            def _():
                pltpu.async_copy(hw2_hbm.at[sidx.at[lax.rem(jj, 8)]],
                                 rows.at[lax.rem(jj, 4)], sem_g)

        def wait_gather(jj):
            @pl.when(valid(jj))
            def _():
                pltpu.make_async_copy(hw2_hbm.at[sidx.at[lax.rem(jj, 8)]],
                                      rows.at[lax.rem(jj, 4)], sem_g).wait()

        def issue_scat(jj):
            @pl.when(valid(jj))
            def _():
                pltpu.async_copy(rows.at[lax.rem(jj, 4)],
                                 acc_sh.at[didx.at[lax.rem(jj, 8)]],
                                 sem_s, add=True)

        def wait_scat(jj):
            @pl.when(valid(jj))
            def _():
                pltpu.make_async_copy(rows.at[lax.rem(jj, 4)],
                                      acc_sh.at[didx.at[lax.rem(jj, 8)]],
                                      sem_s).wait()

        issue_idx(0)
        issue_idx(1)

        @pl.when(cid == 0)
        def _():
            pltpu.sync_copy(hw2_hbm.at[pl.ds(r0, ROWS_PER_TILE)],
                            acc_sh.at[pl.ds(r0, ROWS_PER_TILE)])

        @pl.when(cid != 0)
        def _():
            zv = jnp.zeros((16,), jnp.float32)

            def zb(i, c):
                rows[0, i // 8, pl.ds(lax.rem(i, 8) * 16, 16)] = zv
                return c

            lax.fori_loop(0, KM * 8, zb, 0)
            for t in range(NPAD // NS // KM):
                pltpu.sync_copy(rows.at[0],
                                acc_sh.at[pl.ds(r0 + t * KM, KM)])

        plsc.subcore_barrier()

        def body(j, c):
            wait_scat(j - 4)
            wait_idx(j)
            issue_gather(j)
            wait_gather(j - 2)
            issue_scat(j - 2)
            issue_idx(j + 2)
            return c

        lax.fori_loop(0, JMAXM + 6, body, 0)
        plsc.subcore_barrier()
        pltpu.sync_copy(acc_sh.at[pl.ds(r0, ROWS_PER_TILE)],
                        acc_hbm.at[cid, pl.ds(r0, ROWS_PER_TILE)])

    return deg_kernel, msg_kernel


def _dense_body(x_ref, w_ref, g_ref, bt_ref, degf_ref, hw2_ref, dis_ref):
    x = x_ref[...]
    mean = jnp.mean(x, axis=0, keepdims=True)
    var = jnp.mean((x - mean) ** 2, axis=0, keepdims=True)
    h = (x - mean) * lax.rsqrt(var + 1e-5) * g_ref[...] + bt_ref[...]
    hw = jnp.dot(h, w_ref[...], preferred_element_type=jnp.float32)
    degf = degf_ref[...]
    deg_row = (degf[:NPAD] + degf[NPAD:] + 1.0).reshape(1, NPAD)
    dis = lax.rsqrt(jnp.transpose(deg_row)[:N])
    hw2_ref[pl.ds(0, N)] = hw * dis
    dis_ref[...] = dis


_dense_call = pl.pallas_call(
    _dense_body,
    out_shape=[
        jax.ShapeDtypeStruct((NPAD, D), jnp.float32),
        jax.ShapeDtypeStruct((N, 1), jnp.float32),
    ],
)

BR = 2000


def _out_body(acc_ref, dis_ref, b_ref, x_ref, y_ref):
    s = acc_ref[0] + acc_ref[1]
    o = jnp.maximum(s * dis_ref[...] + b_ref[...], 0.0)
    y_ref[...] = jnp.concatenate([o, x_ref[...]], axis=1)


_out_call = pl.pallas_call(
    _out_body,
    grid=(N // BR,),
    in_specs=[
        pl.BlockSpec((NC, BR, D), lambda i: (0, i, 0)),
        pl.BlockSpec((BR, 1), lambda i: (i, 0)),
        pl.BlockSpec((D,), lambda i: (0,)),
        pl.BlockSpec((BR, D), lambda i: (i, 0)),
    ],
    out_specs=pl.BlockSpec((BR, 2 * D), lambda i: (i, 0)),
    out_shape=jax.ShapeDtypeStruct((N, 2 * D), jnp.float32),
)


def kernel(x, edge_index, gamma, beta, W, b):
    deg_call, msg_call = _sc_kernels()
    ei = edge_index.astype(jnp.int32).reshape(2 * E)
    degf = deg_call(ei)
    hw2, dis = _dense_call(x, W, gamma, beta, degf)
    acc = msg_call(ei, hw2)
    return _out_call(acc, dis, b, x)


# speedup vs baseline: 47.2615x; 1.0121x over previous
import functools

import jax
import jax.numpy as jnp
from jax import lax
from jax.experimental import pallas as pl
from jax.experimental.pallas import tpu as pltpu
from jax.experimental.pallas import tpu_sc as plsc

N = 10000
D = 128
E = 320000
NC = 2
NS = 16
NW = NC * NS
KD = 256
NCHUNKD = E // KD
JMAXD = (NCHUNKD + NW - 1) // NW
KM = 80
NCHUNKM = E // KM
JMAXM = NCHUNKM // NW
NPAD = 10240
DEG_PER_TILE = NPAD // NS
ROWS_PER_TILE = NPAD // NS


def _fill_f32(ref, nwords, value):
    v = jnp.full((16,), value, jnp.float32)

    def body(i, c):
        ref[pl.ds(i * 16, 16)] = v
        return c

    lax.fori_loop(0, nwords // 16, body, 0)


@functools.cache
def _sc_kernels():
    mesh = plsc.VectorSubcoreMesh(core_axis_name="c", subcore_axis_name="s",
                                  num_cores=NC, num_subcores=NS)

    @functools.partial(
        pl.kernel,
        out_type=jax.ShapeDtypeStruct((NC * NPAD,), jnp.float32),
        mesh=mesh,
        scratch_types=[
            pltpu.VMEM((8, 2, 128), jnp.int32),
            pltpu.VMEM((128,), jnp.float32),
            pltpu.VMEM((DEG_PER_TILE,), jnp.float32),
            pltpu.VMEM_SHARED((NPAD,), jnp.float32),
            pltpu.SemaphoreType.DMA,
            pltpu.SemaphoreType.DMA,
        ],
    )
    def deg_kernel(ei_hbm, deg_hbm, didx, ones_v, zero_v, deg_sh,
                   sem_i, sem_s):
        cid = lax.axis_index("c")
        sid = lax.axis_index("s")
        wid = sid * NC + cid
        d0 = sid * DEG_PER_TILE
        _fill_f32(zero_v, DEG_PER_TILE, 0.0)
        _fill_f32(ones_v, 128, 1.0)
        pltpu.sync_copy(zero_v, deg_sh.at[pl.ds(d0, DEG_PER_TILE)])
        plsc.subcore_barrier()

        def valid(jj):
            return (jj >= 0) & (wid + jj * NW < NCHUNKD)

        def issue_idx(jj):
            @pl.when(valid(jj))
            def _():
                base = E + (wid + jj * NW) * KD
                s = lax.rem(jj, 8)
                pltpu.async_copy(ei_hbm.at[pl.ds(base, 128)],
                                 didx.at[s, 0], sem_i)
                pltpu.async_copy(ei_hbm.at[pl.ds(base + 128, 128)],
                                 didx.at[s, 1], sem_i)

        def wait_idx(jj):
            @pl.when(valid(jj))
            def _():
                base = E + (wid + jj * NW) * KD
                s = lax.rem(jj, 8)
                pltpu.make_async_copy(ei_hbm.at[pl.ds(base, 128)],
                                      didx.at[s, 0], sem_i).wait()
                pltpu.make_async_copy(ei_hbm.at[pl.ds(base + 128, 128)],
                                      didx.at[s, 1], sem_i).wait()

        def issue_scat(jj):
            @pl.when(valid(jj))
            def _():
                s = lax.rem(jj, 8)
                pltpu.async_copy(ones_v, deg_sh.at[didx.at[s, 0]],
                                 sem_s, add=True)
                pltpu.async_copy(ones_v, deg_sh.at[didx.at[s, 1]],
                                 sem_s, add=True)

        def wait_scat(jj):
            @pl.when(valid(jj))
            def _():
                s = lax.rem(jj, 8)
                pltpu.make_async_copy(ones_v, deg_sh.at[didx.at[s, 0]],
                                      sem_s).wait()
                pltpu.make_async_copy(ones_v, deg_sh.at[didx.at[s, 1]],
                                      sem_s).wait()

        issue_idx(0)
        issue_idx(1)

        def body(j, c):
            wait_scat(j - 4)
            wait_idx(j)
            issue_scat(j)
            issue_idx(j + 2)
            return c

        lax.fori_loop(0, JMAXD + 4, body, 0)
        plsc.subcore_barrier()
        pltpu.sync_copy(deg_sh.at[pl.ds(d0, DEG_PER_TILE)],
                        deg_hbm.at[pl.ds(cid * NPAD + d0, DEG_PER_TILE)])

    @functools.partial(
        pl.kernel,
        out_type=jax.ShapeDtypeStruct((NC, NPAD, D), jnp.float32),
        mesh=mesh,
        scratch_types=[
            pltpu.VMEM((8, KM), jnp.int32),
            pltpu.VMEM((8, KM), jnp.int32),
            pltpu.VMEM((4, KM, D), jnp.float32),
            pltpu.VMEM_SHARED((NPAD, D), jnp.float32),
            pltpu.SemaphoreType.DMA,
            pltpu.SemaphoreType.DMA,
            pltpu.SemaphoreType.DMA,
        ],
    )
    def msg_kernel(ei_hbm, hw2_hbm, acc_hbm,
                   sidx, didx, rows, acc_sh, sem_i, sem_g, sem_s):
        cid = lax.axis_index("c")
        sid = lax.axis_index("s")
        wid = sid * NC + cid
        r0 = sid * ROWS_PER_TILE

        def valid(jj):
            return (jj >= 0) & (wid + jj * NW < NCHUNKM)

        def issue_idx(jj):
            @pl.when(valid(jj))
            def _():
                base = (wid + jj * NW) * KM
                s = lax.rem(jj, 8)
                pltpu.async_copy(ei_hbm.at[pl.ds(base, KM)],
                                 sidx.at[s], sem_i)
                pltpu.async_copy(ei_hbm.at[pl.ds(E + base, KM)],
                                 didx.at[s], sem_i)

        def wait_idx(jj):
            @pl.when(valid(jj))
            def _():
                base = (wid + jj * NW) * KM
                s = lax.rem(jj, 8)
                pltpu.make_async_copy(ei_hbm.at[pl.ds(base, KM)],
                                      sidx.at[s], sem_i).wait()
                pltpu.make_async_copy(ei_hbm.at[pl.ds(E + base, KM)],
                                      didx.at[s], sem_i).wait()

        def issue_gather(jj):
            @pl.when(valid(jj))
            def _():
                pltpu.async_copy(hw2_hbm.at[sidx.at[lax.rem(jj, 8)]],
                                 rows.at[lax.rem(jj, 4)], sem_g)

        def wait_gather(jj):
            @pl.when(valid(jj))
            def _():
                pltpu.make_async_copy(hw2_hbm.at[sidx.at[lax.rem(jj, 8)]],
                                      rows.at[lax.rem(jj, 4)], sem_g).wait()

        def issue_scat(jj):
            @pl.when(valid(jj))
            def _():
                pltpu.async_copy(rows.at[lax.rem(jj, 4)],
                                 acc_sh.at[didx.at[lax.rem(jj, 8)]],
                                 sem_s, add=True)

        def wait_scat(jj):
            @pl.when(valid(jj))
            def _():
                pltpu.make_async_copy(rows.at[lax.rem(jj, 4)],
                                      acc_sh.at[didx.at[lax.rem(jj, 8)]],
                                      sem_s).wait()

        issue_idx(0)
        issue_idx(1)

        @pl.when(cid == 0)
        def _():
            pltpu.sync_copy(hw2_hbm.at[pl.ds(r0, ROWS_PER_TILE)],
                            acc_sh.at[pl.ds(r0, ROWS_PER_TILE)])

        @pl.when(cid != 0)
        def _():
            zv = jnp.zeros((16,), jnp.float32)

            def zb(i, c):
                rows[0, i // 8, pl.ds(lax.rem(i, 8) * 16, 16)] = zv
                return c

            lax.fori_loop(0, KM * 8, zb, 0)
            for t in range(NPAD // NS // KM):
                pltpu.sync_copy(rows.at[0],
                                acc_sh.at[pl.ds(r0 + t * KM, KM)])

        plsc.subcore_barrier()

        def body(j, c):
            wait_scat(j - 4)
            wait_idx(j)
            issue_gather(j)
            wait_gather(j - 2)
            issue_scat(j - 2)
            issue_idx(j + 2)
            return c

        lax.fori_loop(0, JMAXM + 6, body, 0)
        plsc.subcore_barrier()
        pltpu.sync_copy(acc_sh.at[pl.ds(r0, ROWS_PER_TILE)],
                        acc_hbm.at[cid, pl.ds(r0, ROWS_PER_TILE)])

    return deg_kernel, msg_kernel


def _dense_body(x_ref, w_ref, g_ref, bt_ref, degf_ref, hw2_ref, dis_ref):
    x = x_ref[...]
    mean = jnp.mean(x, axis=0, keepdims=True)
    var = jnp.mean((x - mean) ** 2, axis=0, keepdims=True)
    h = (x - mean) * lax.rsqrt(var + 1e-5) * g_ref[...] + bt_ref[...]
    hw = jnp.dot(h, w_ref[...], preferred_element_type=jnp.float32)
    degf = degf_ref[...]
    deg_row = (degf[:NPAD] + degf[NPAD:] + 1.0).reshape(1, NPAD)
    dis = lax.rsqrt(jnp.transpose(deg_row)[:N])
    hw2_ref[pl.ds(0, N)] = hw * dis
    dis_ref[...] = dis


_dense_call = pl.pallas_call(
    _dense_body,
    out_shape=[
        jax.ShapeDtypeStruct((NPAD, D), jnp.float32),
        jax.ShapeDtypeStruct((N, 1), jnp.float32),
    ],
)

def _flat_body(ei_ref, out_ref):
    out_ref[pl.ds(0, E)] = ei_ref[0]
    out_ref[pl.ds(E, E)] = ei_ref[1]


_flat_call = pl.pallas_call(
    _flat_body,
    out_shape=jax.ShapeDtypeStruct((2 * E,), jnp.int32),
)


BR = 2000


def _out_body(acc_ref, dis_ref, b_ref, x_ref, y_ref):
    s = acc_ref[0] + acc_ref[1]
    o = jnp.maximum(s * dis_ref[...] + b_ref[...], 0.0)
    y_ref[...] = jnp.concatenate([o, x_ref[...]], axis=1)


_out_call = pl.pallas_call(
    _out_body,
    grid=(N // BR,),
    in_specs=[
        pl.BlockSpec((NC, BR, D), lambda i: (0, i, 0)),
        pl.BlockSpec((BR, 1), lambda i: (i, 0)),
        pl.BlockSpec((D,), lambda i: (0,)),
        pl.BlockSpec((BR, D), lambda i: (i, 0)),
    ],
    out_specs=pl.BlockSpec((BR, 2 * D), lambda i: (i, 0)),
    out_shape=jax.ShapeDtypeStruct((N, 2 * D), jnp.float32),
)


def kernel(x, edge_index, gamma, beta, W, b):
    deg_call, msg_call = _sc_kernels()
    ei = _flat_call(edge_index.astype(jnp.int32))
    degf = deg_call(ei)
    hw2, dis = _dense_call(x, W, gamma, beta, degf)
    acc = msg_call(ei, hw2)
    return _out_call(acc, dis, b, x)
